# Initial kernel scaffold; baseline (speedup 1.0000x reference)
import jax
import jax.numpy as jnp
from jax.experimental import pallas as pl


def kernel(label_lists, table):
    raise NotImplementedError("write your pallas kernel here")


# speedup vs baseline: 2.5166x; 2.5166x over previous
import functools

import jax
import jax.numpy as jnp
from jax import lax
from jax.experimental import pallas as pl
from jax.experimental.pallas import tpu as pltpu
from jax.experimental.pallas import tpu_sc as plsc

B = 16384
L = 50
D = 32
NL = 16

_info = plsc.get_sparse_core_info()
NC = _info.num_cores
NS = _info.num_subcores
NW = NC * NS
B_PER_W = B // NW
CH = 32
N_CHUNKS = B_PER_W // CH


def _body(labels_hbm, table_hbm, out_hbm, idx_v, rows_v, out_v, sem):
    wid = lax.axis_index("s") * NC + lax.axis_index("c")
    base_row = wid * B_PER_W
    scale = jnp.float32(1.0 / L)

    def chunk_body(c, carry):
        row0 = base_row + c * CH
        pltpu.sync_copy(labels_hbm.at[pl.ds(row0 * L, CH * L)], idx_v)
        pltpu.async_copy(table_hbm.at[idx_v], rows_v, sem).wait()

        def row_body(r, carry2):
            def acc_body(j, accs):
                a0, a1 = accs
                g = r * L + j
                a0 = a0 + rows_v[g, pl.ds(0, NL)]
                a1 = a1 + rows_v[g, pl.ds(NL, NL)]
                return (a0, a1)

            zero = jnp.zeros((NL,), jnp.float32)
            a0, a1 = lax.fori_loop(0, L, acc_body, (zero, zero))
            out_v[r, pl.ds(0, NL)] = a0 * scale
            out_v[r, pl.ds(NL, NL)] = a1 * scale
            return carry2

        lax.fori_loop(0, CH, row_body, 0)
        pltpu.sync_copy(out_v, out_hbm.at[pl.ds(row0, CH)])
        return carry

    lax.fori_loop(0, N_CHUNKS, chunk_body, 0)


def kernel(label_lists, table):
    labels_flat = label_lists.reshape(-1).astype(jnp.int32)
    mesh = plsc.VectorSubcoreMesh(core_axis_name="c", subcore_axis_name="s")
    k = pl.kernel(
        _body,
        mesh=mesh,
        out_type=jax.ShapeDtypeStruct((B, D), jnp.float32),
        scratch_types=[
            pltpu.VMEM((CH * L,), jnp.int32),
            pltpu.VMEM((CH * L, D), jnp.float32),
            pltpu.VMEM((CH, D), jnp.float32),
            pltpu.SemaphoreType.DMA,
        ],
        compiler_params=pltpu.CompilerParams(use_tc_tiling_on_sc=False),
    )
    return k(labels_flat, table)


# speedup vs baseline: 2.9432x; 1.1695x over previous
import jax
import jax.numpy as jnp
from jax import lax
from jax.experimental import pallas as pl
from jax.experimental.pallas import tpu as pltpu
from jax.experimental.pallas import tpu_sc as plsc

B = 16384
L = 50
D = 32
NL = 16
NACC = 4

_info = plsc.get_sparse_core_info()
NC = _info.num_cores
NS = _info.num_subcores
NW = NC * NS
B_PER_W = B // NW
CH = 32
N_CHUNKS = B_PER_W // CH


def _body(labels_hbm, table_hbm, out_hbm,
          idx0, idx1, rows0, rows1, outv0, outv1,
          gsem0, gsem1, osem0, osem1):
    wid = lax.axis_index("s") * NC + lax.axis_index("c")
    base_row = wid * B_PER_W
    scale = jnp.float32(1.0 / L)
    idx_v = (idx0, idx1)
    rows_v = (rows0, rows1)
    out_v = (outv0, outv1)
    gsem = (gsem0, gsem1)
    osem = (osem0, osem1)

    def stage_and_fire(c, b):
        row0 = base_row + c * CH
        pltpu.sync_copy(labels_hbm.at[pl.ds(row0 * L, CH * L)], idx_v[b])
        pltpu.async_copy(table_hbm.at[idx_v[b]], rows_v[b], gsem[b])

    def compute_chunk(c, b):
        row0 = base_row + c * CH
        pltpu.make_async_copy(table_hbm.at[idx_v[b]], rows_v[b], gsem[b]).wait()

        def row_body(r, carry):
            g0 = r * L
            acc = [jnp.zeros((NL,), jnp.float32) for _ in range(2 * NACC)]
            for j in range(L):
                k = j % NACC
                acc[k] = acc[k] + rows_v[b][g0 + j, pl.ds(0, NL)]
                acc[NACC + k] = acc[NACC + k] + rows_v[b][g0 + j, pl.ds(NL, NL)]
            lo = (acc[0] + acc[1]) + (acc[2] + acc[3])
            hi = (acc[4] + acc[5]) + (acc[6] + acc[7])
            out_v[b][r, pl.ds(0, NL)] = lo * scale
            out_v[b][r, pl.ds(NL, NL)] = hi * scale
            return carry

        lax.fori_loop(0, CH, row_body, 0)
        pltpu.async_copy(out_v[b], out_hbm.at[pl.ds(row0, CH)], osem[b])

    stage_and_fire(0, 0)

    def loop_body(c2, carry):
        for bb in range(2):
            c = c2 * 2 + bb

            @pl.when(c + 1 < N_CHUNKS)
            def _():
                stage_and_fire(c + 1, 1 - bb)

            @pl.when(c >= 2)
            def _():
                pltpu.make_async_copy(
                    out_v[bb], out_hbm.at[pl.ds(base_row + c * CH, CH)],
                    osem[bb]).wait()

            compute_chunk(c, bb)
        return carry

    lax.fori_loop(0, N_CHUNKS // 2, loop_body, 0)

    for bb in range(2):
        pltpu.make_async_copy(
            out_v[bb], out_hbm.at[pl.ds(base_row, CH)], osem[bb]).wait()


def kernel(label_lists, table):
    labels_flat = label_lists.reshape(-1).astype(jnp.int32)
    mesh = plsc.VectorSubcoreMesh(core_axis_name="c", subcore_axis_name="s")
    k = pl.kernel(
        _body,
        mesh=mesh,
        out_type=jax.ShapeDtypeStruct((B, D), jnp.float32),
        scratch_types=[
            pltpu.VMEM((CH * L,), jnp.int32),
            pltpu.VMEM((CH * L,), jnp.int32),
            pltpu.VMEM((CH * L, D), jnp.float32),
            pltpu.VMEM((CH * L, D), jnp.float32),
            pltpu.VMEM((CH, D), jnp.float32),
            pltpu.VMEM((CH, D), jnp.float32),
            pltpu.SemaphoreType.DMA,
            pltpu.SemaphoreType.DMA,
            pltpu.SemaphoreType.DMA,
            pltpu.SemaphoreType.DMA,
        ],
        compiler_params=pltpu.CompilerParams(use_tc_tiling_on_sc=False),
    )
    return k(labels_flat, table)
